# Initial kernel scaffold; baseline (speedup 1.0000x reference)
import jax
import jax.numpy as jnp
from jax.experimental import pallas as pl


def kernel(x, glove_weight, fc_w, fc_b):
    raise NotImplementedError("write your pallas kernel here")


# speedup vs baseline: 2.5025x; 2.5025x over previous
import functools

import jax
import jax.numpy as jnp
from jax import lax
from jax.experimental import pallas as pl
from jax.experimental.pallas import tpu as pltpu
from jax.experimental.pallas import tpu_sc as plsc

BATCH = 16384
HIST = 50
EMBED_DIM = 64
NUM_CLASSES = 2

NC = 2
NS = 16
NW = NC * NS
LANES = 16

SAMPLES_PER_W = BATCH // NW
CHUNK_SAMPLES = 8
CHUNK_ROWS = CHUNK_SAMPLES * HIST
GATHERS_PER_CHUNK = 4
IDX_PER_GATHER = CHUNK_ROWS // GATHERS_PER_CHUNK
NCHUNKS = SAMPLES_PER_W // CHUNK_SAMPLES
IDX_ROWS_PER_W = SAMPLES_PER_W * HIST // IDX_PER_GATHER


def _sc_body(table_hbm, idx_hbm, fcwt_hbm, bias_hbm, out_hbm,
             idx_v, rows_v, fcw_v, bias_v, out_v, sem):
    wid = lax.axis_index("s") * NC + lax.axis_index("c")

    pltpu.sync_copy(idx_hbm.at[pl.ds(wid * IDX_ROWS_PER_W, IDX_ROWS_PER_W)],
                    idx_v)
    pltpu.sync_copy(fcwt_hbm, fcw_v)
    pltpu.sync_copy(bias_hbm, bias_v)

    w0 = [fcw_v[0, pl.ds(j * LANES, LANES)] for j in range(4)]
    w1 = [fcw_v[1, pl.ds(j * LANES, LANES)] for j in range(4)]
    bias = bias_v[pl.ds(0, LANES)]
    li = lax.iota(jnp.int32, LANES)
    inv_hist = jnp.float32(1.0 / HIST)
    rots = [(li + k) & (LANES - 1) for k in (8, 4, 2, 1)]

    gdn = lax.GatherDimensionNumbers(
        offset_dims=(), collapsed_slice_dims=(0,), start_index_map=(0,))

    def lane_take(v, r):
        return lax.gather(v, r[:, None], dimension_numbers=gdn,
                          slice_sizes=(1,),
                          mode=lax.GatherScatterMode.PROMISE_IN_BOUNDS)

    def allsum(v):
        for r in rots:
            v = v + lane_take(v, r)
        return v

    def chunk_body(c, carry):
        handles = [
            pltpu.async_copy(
                table_hbm.at[idx_v.at[GATHERS_PER_CHUNK * c + k]],
                rows_v.at[0, pl.ds(k * IDX_PER_GATHER, IDX_PER_GATHER)],
                sem)
            for k in range(GATHERS_PER_CHUNK)
        ]
        for h in handles:
            h.wait()

        ov = bias
        for s in range(CHUNK_SAMPLES):
            def row_body(r, accs):
                row = s * HIST + r
                return tuple(
                    accs[j] + rows_v[0, row, pl.ds(j * LANES, LANES)]
                    for j in range(4))
            z = jnp.zeros((LANES,), jnp.float32)
            a = lax.fori_loop(0, HIST, row_body, (z, z, z, z), unroll=5)
            t0 = a[0] * w0[0] + a[1] * w0[1] + a[2] * w0[2] + a[3] * w0[3]
            t1 = a[0] * w1[0] + a[1] * w1[1] + a[2] * w1[2] + a[3] * w1[3]
            p0 = allsum(t0) * inv_hist
            p1 = allsum(t1) * inv_hist
            ov = ov + jnp.where(li == 2 * s, p0, 0.0)
            ov = ov + jnp.where(li == 2 * s + 1, p1, 0.0)
        out_v[pl.ds(c * LANES, LANES)] = ov
        return carry

    lax.fori_loop(0, NCHUNKS, chunk_body, 0)

    pltpu.sync_copy(
        out_v, out_hbm.at[pl.ds(wid * SAMPLES_PER_W * NUM_CLASSES,
                                SAMPLES_PER_W * NUM_CLASSES)])


@jax.jit
def kernel(x, glove_weight, fc_w, fc_b):
    idx = x.reshape(-1, IDX_PER_GATHER).astype(jnp.int32)
    fc_wt = fc_w.T.reshape(NUM_CLASSES, EMBED_DIM)
    bias_tile = jnp.tile(fc_b.astype(jnp.float32), LANES // NUM_CLASSES)

    mesh = plsc.VectorSubcoreMesh(core_axis_name="c", subcore_axis_name="s")
    run = pl.kernel(
        _sc_body,
        out_type=jax.ShapeDtypeStruct((BATCH * NUM_CLASSES,), jnp.float32),
        mesh=mesh,
        scratch_types=[
            pltpu.VMEM((IDX_ROWS_PER_W, IDX_PER_GATHER), jnp.int32),
            pltpu.VMEM((1, CHUNK_ROWS, EMBED_DIM), jnp.float32),
            pltpu.VMEM((NUM_CLASSES, EMBED_DIM), jnp.float32),
            pltpu.VMEM((LANES,), jnp.float32),
            pltpu.VMEM((SAMPLES_PER_W * NUM_CLASSES,), jnp.float32),
            pltpu.SemaphoreType.DMA,
        ],
        compiler_params=pltpu.CompilerParams(use_tc_tiling_on_sc=False),
    )
    out_flat = run(glove_weight, idx, fc_wt, bias_tile)
    return out_flat.reshape(BATCH, NUM_CLASSES)


# speedup vs baseline: 2.5153x; 1.0051x over previous
import functools

import jax
import jax.numpy as jnp
from jax import lax
from jax.experimental import pallas as pl
from jax.experimental.pallas import tpu as pltpu
from jax.experimental.pallas import tpu_sc as plsc

BATCH = 16384
HIST = 50
EMBED_DIM = 64
NUM_CLASSES = 2

NC = 2
NS = 16
NW = NC * NS
LANES = 16

SAMPLES_PER_W = BATCH // NW
CHUNK_SAMPLES = 8
CHUNK_ROWS = CHUNK_SAMPLES * HIST
GATHERS_PER_CHUNK = 1
IDX_PER_GATHER = CHUNK_ROWS // GATHERS_PER_CHUNK
NCHUNKS = SAMPLES_PER_W // CHUNK_SAMPLES
IDX_ROWS_PER_W = SAMPLES_PER_W * HIST // IDX_PER_GATHER


def _sc_body(table_hbm, idx_hbm, fcwt_hbm, bias_hbm, out_hbm,
             idx_v, rows_v, fcw_v, bias_v, out_v, sem):
    wid = lax.axis_index("s") * NC + lax.axis_index("c")

    pltpu.sync_copy(idx_hbm.at[pl.ds(wid * IDX_ROWS_PER_W, IDX_ROWS_PER_W)],
                    idx_v)
    pltpu.sync_copy(fcwt_hbm, fcw_v)
    pltpu.sync_copy(bias_hbm, bias_v)

    w0 = [fcw_v[0, pl.ds(j * LANES, LANES)] for j in range(4)]
    w1 = [fcw_v[1, pl.ds(j * LANES, LANES)] for j in range(4)]
    bias = bias_v[pl.ds(0, LANES)]
    li = lax.iota(jnp.int32, LANES)
    inv_hist = jnp.float32(1.0 / HIST)
    rots = [(li + k) & (LANES - 1) for k in (8, 4, 2, 1)]

    gdn = lax.GatherDimensionNumbers(
        offset_dims=(), collapsed_slice_dims=(0,), start_index_map=(0,))

    def lane_take(v, r):
        return lax.gather(v, r[:, None], dimension_numbers=gdn,
                          slice_sizes=(1,),
                          mode=lax.GatherScatterMode.PROMISE_IN_BOUNDS)

    def allsum(v):
        for r in rots:
            v = v + lane_take(v, r)
        return v

    def chunk_body(c, carry):
        handles = [
            pltpu.async_copy(
                table_hbm.at[idx_v.at[GATHERS_PER_CHUNK * c + k]],
                rows_v.at[0, pl.ds(k * IDX_PER_GATHER, IDX_PER_GATHER)],
                sem)
            for k in range(GATHERS_PER_CHUNK)
        ]
        for h in handles:
            h.wait()

        ov = bias
        for s in range(CHUNK_SAMPLES):
            def row_body(r, accs):
                row = s * HIST + r
                return tuple(
                    accs[j] + rows_v[0, row, pl.ds(j * LANES, LANES)]
                    for j in range(4))
            z = jnp.zeros((LANES,), jnp.float32)
            a = lax.fori_loop(0, HIST, row_body, (z, z, z, z), unroll=5)
            t0 = a[0] * w0[0] + a[1] * w0[1] + a[2] * w0[2] + a[3] * w0[3]
            t1 = a[0] * w1[0] + a[1] * w1[1] + a[2] * w1[2] + a[3] * w1[3]
            p0 = allsum(t0) * inv_hist
            p1 = allsum(t1) * inv_hist
            ov = ov + jnp.where(li == 2 * s, p0, 0.0)
            ov = ov + jnp.where(li == 2 * s + 1, p1, 0.0)
        out_v[pl.ds(c * LANES, LANES)] = ov
        return carry

    lax.fori_loop(0, NCHUNKS, chunk_body, 0)

    pltpu.sync_copy(
        out_v, out_hbm.at[pl.ds(wid * SAMPLES_PER_W * NUM_CLASSES,
                                SAMPLES_PER_W * NUM_CLASSES)])


@jax.jit
def kernel(x, glove_weight, fc_w, fc_b):
    idx = x.reshape(-1, IDX_PER_GATHER).astype(jnp.int32)
    fc_wt = fc_w.T.reshape(NUM_CLASSES, EMBED_DIM)
    bias_tile = jnp.tile(fc_b.astype(jnp.float32), LANES // NUM_CLASSES)

    mesh = plsc.VectorSubcoreMesh(core_axis_name="c", subcore_axis_name="s")
    run = pl.kernel(
        _sc_body,
        out_type=jax.ShapeDtypeStruct((BATCH * NUM_CLASSES,), jnp.float32),
        mesh=mesh,
        scratch_types=[
            pltpu.VMEM((IDX_ROWS_PER_W, IDX_PER_GATHER), jnp.int32),
            pltpu.VMEM((1, CHUNK_ROWS, EMBED_DIM), jnp.float32),
            pltpu.VMEM((NUM_CLASSES, EMBED_DIM), jnp.float32),
            pltpu.VMEM((LANES,), jnp.float32),
            pltpu.VMEM((SAMPLES_PER_W * NUM_CLASSES,), jnp.float32),
            pltpu.SemaphoreType.DMA,
        ],
        compiler_params=pltpu.CompilerParams(use_tc_tiling_on_sc=False),
    )
    out_flat = run(glove_weight, idx, fc_wt, bias_tile)
    return out_flat.reshape(BATCH, NUM_CLASSES)
